# baseline (speedup 1.0000x reference)
import functools

import jax
import jax.numpy as jnp
from jax import lax
from jax.experimental import pallas as pl
from jax.experimental.pallas import tpu as pltpu
from jax.experimental.pallas import tpu_sc as plsc

K = 1024
DIM = 256
MOMENTUM = 0.1
LANES = 16
NUM_CORES = 2
NUM_SUBCORES = 16
NUM_WORKERS = NUM_CORES * NUM_SUBCORES
ROWS_PER_W = K // NUM_WORKERS
NVEC = DIM // LANES


_GATHER_DNUMS = lax.GatherDimensionNumbers(
    offset_dims=(), collapsed_slice_dims=(0,), start_index_map=(0,))


def _shuffle(v, idx):
    return lax.gather(v, idx.reshape(LANES, 1), _GATHER_DNUMS, (1,),
                      mode=lax.GatherScatterMode.PROMISE_IN_BOUNDS)


def _lane_sum(v):
    lanes = lax.iota(jnp.int32, LANES)
    for k in (8, 4, 2, 1):
        v = v + _shuffle(v, lanes ^ k)
    return v


def _rsqrt_nr(s):
    bits = lax.bitcast_convert_type(s, jnp.int32)
    y = lax.bitcast_convert_type(jnp.int32(0x5F3759DF) - (bits >> 1), jnp.float32)
    for _ in range(3):
        y = y * (1.5 - 0.5 * s * y * y)
    return y


@functools.cache
def _build_update():
    mesh = plsc.VectorSubcoreMesh(
        core_axis_name="c", subcore_axis_name="s",
        num_cores=NUM_CORES, num_subcores=NUM_SUBCORES)

    @functools.partial(
        pl.kernel,
        out_type=jax.ShapeDtypeStruct((K, DIM), jnp.float32),
        mesh=mesh,
        scratch_types=[
            pltpu.VMEM((ROWS_PER_W, DIM), jnp.float32),
            pltpu.VMEM((ROWS_PER_W, DIM), jnp.float32),
            pltpu.VMEM((ROWS_PER_W, DIM), jnp.float32),
        ],
    )
    def update(att_hbm, cent_hbm, out_hbm, att_v, cent_v, out_v):
        wid = lax.axis_index("s") * NUM_CORES + lax.axis_index("c")
        base = wid * ROWS_PER_W
        pltpu.sync_copy(att_hbm.at[pl.ds(base, ROWS_PER_W)], att_v)
        pltpu.sync_copy(cent_hbm.at[pl.ds(base, ROWS_PER_W)], cent_v)

        def row(r, carry):
            u = []
            acc = jnp.zeros((LANES,), jnp.float32)
            for j in range(NVEC):
                a = att_v[r, pl.ds(j * LANES, LANES)]
                c = cent_v[r, pl.ds(j * LANES, LANES)]
                v = (1.0 - MOMENTUM) * a + MOMENTUM * c
                u.append(v)
                acc = acc + v * v
            s = _lane_sum(acc)
            norm = s * _rsqrt_nr(s)
            inv = 1.0 / jnp.maximum(norm, 1e-12)
            for j in range(NVEC):
                out_v[r, pl.ds(j * LANES, LANES)] = u[j] * inv
            return carry

        lax.fori_loop(0, ROWS_PER_W, row, 0)
        pltpu.sync_copy(out_v, out_hbm.at[pl.ds(base, ROWS_PER_W)])

    return update


def kernel(new_centroids, cluster_counts, position_type, attractors_0,
           attractors_1, attractors_2):
    del cluster_counts
    update = _build_update()
    pt = jnp.asarray(position_type, jnp.int32)
    return lax.switch(pt, [
        lambda nc: update(attractors_0, nc),
        lambda nc: update(attractors_1, nc),
        lambda nc: update(attractors_2, nc),
    ], new_centroids)
